# baseline (speedup 1.0000x reference)
import functools

import jax
import jax.numpy as jnp
from jax import lax
from jax.experimental import pallas as pl
from jax.experimental.pallas import tpu as pltpu
from jax.experimental.pallas import tpu_sc as plsc

_BATCH = 16384
_N = 256
_L = 16
_NG = _N // _L
_NC = 1
_NS = 16
_NW = _NC * _NS

_F_SC = 1024
_RPW = _F_SC // _NW
_CH = 32
_NCHUNK = _RPW // _CH
_NPAIR = _NCHUNK // 2

_BLK = 1024
_TC_ROWS = _BATCH - _F_SC


def _sc_body(x_hbm, w_hbm, no_hbm, out_hbm,
             no_v, w_v, xin0, xin1, xout0, xout1,
             sin0, sin1, sout0, sout1):
    wid = lax.axis_index("s") * _NC + lax.axis_index("c")
    base = wid * _RPW
    last = base + _RPW - _CH

    pltpu.sync_copy(no_hbm, no_v)
    pltpu.sync_copy(w_hbm, w_v)
    no_g = [no_v[pl.ds(g * _L, _L)] for g in range(_NG)]
    w_g = [w_v[pl.ds(g * _L, _L)] for g in range(_NG)]

    def start_in(buf, sem, row0):
        pltpu.async_copy(x_hbm.at[pl.ds(row0, _CH)], buf, sem)

    def wait_in(buf, sem):
        pltpu.make_async_copy(x_hbm.at[pl.ds(0, _CH)], buf, sem).wait()

    def start_out(buf, sem, row0):
        pltpu.async_copy(buf, out_hbm.at[pl.ds(row0, _CH)], sem)

    def wait_out(buf, sem):
        pltpu.make_async_copy(buf, out_hbm.at[pl.ds(0, _CH)], sem).wait()

    def compute(xi, xo):
        def row_body(r):
            rvec = jnp.full((_L,), r, jnp.int32)
            for g in range(_NG):
                vals = plsc.load_gather(xi, [rvec, no_g[g]])
                xo[r, pl.ds(g * _L, _L)] = vals * w_g[g]
        plsc.parallel_loop(0, _CH, unroll=1)(row_body)

    start_in(xin0, sin0, base)

    def pair_body(i, _):
        c0 = base + (2 * i) * _CH
        c1 = c0 + _CH
        start_in(xin1, sin1, c1)
        wait_in(xin0, sin0)

        @pl.when(i > 0)
        def _():
            wait_out(xout0, sout0)

        compute(xin0, xout0)
        start_out(xout0, sout0, c0)
        start_in(xin0, sin0, jnp.minimum(c1 + _CH, last))
        wait_in(xin1, sin1)

        @pl.when(i > 0)
        def _():
            wait_out(xout1, sout1)

        compute(xin1, xout1)
        start_out(xout1, sout1, c1)
        return 0

    lax.fori_loop(0, _NPAIR, pair_body, 0)

    wait_in(xin0, sin0)
    wait_out(xout0, sout0)
    wait_out(xout1, sout1)


def _sc_call(x, weights, node_order):
    mesh = plsc.VectorSubcoreMesh(core_axis_name="c", subcore_axis_name="s", num_cores=1)
    k = functools.partial(
        pl.kernel,
        mesh=mesh,
        out_type=jax.ShapeDtypeStruct((_F_SC, _N), jnp.float32),
        compiler_params=pltpu.CompilerParams(needs_layout_passes=False),
        scratch_types=[
            pltpu.VMEM((_N,), jnp.int32),
            pltpu.VMEM((_N,), jnp.float32),
            pltpu.VMEM((_CH, _N), jnp.float32),
            pltpu.VMEM((_CH, _N), jnp.float32),
            pltpu.VMEM((_CH, _N), jnp.float32),
            pltpu.VMEM((_CH, _N), jnp.float32),
            pltpu.SemaphoreType.DMA,
            pltpu.SemaphoreType.DMA,
            pltpu.SemaphoreType.DMA,
            pltpu.SemaphoreType.DMA,
        ],
    )(_sc_body)
    return k(x, weights, node_order)


def _tc_body(no_ref, w_ref, x_ref, o_ref, p_ref):
    @pl.when(pl.program_id(0) == 0)
    def _():
        ids = lax.broadcasted_iota(jnp.int32, (_N, _N), 0)
        p_ref[...] = jnp.where(ids == no_ref[...], w_ref[...],
                               jnp.float32(0.0))

    o_ref[...] = lax.dot_general(
        x_ref[...], p_ref[...], (((1,), (0,)), ((), ())),
        preferred_element_type=jnp.float32)


def _tc_call(x, no2d, w2d):
    nblk = _TC_ROWS // _BLK
    off = _F_SC // _BLK
    return pl.pallas_call(
        _tc_body,
        grid=(nblk,),
        in_specs=[
            pl.BlockSpec((1, _N), lambda i: (0, 0)),
            pl.BlockSpec((1, _N), lambda i: (0, 0)),
            pl.BlockSpec((_BLK, _N), lambda i: (i + off, 0)),
        ],
        out_specs=pl.BlockSpec((_BLK, _N), lambda i: (i + off, 0)),
        out_shape=jax.ShapeDtypeStruct((_BATCH, _N), jnp.float32),
        scratch_shapes=[pltpu.VMEM((_N, _N), jnp.float32)],
    )(no2d, w2d, x)


@jax.jit
def _run(x, weights, node_order):
    sc_out = _sc_call(x, weights, node_order)
    tc_full = _tc_call(x, node_order.reshape(1, _N), weights.reshape(1, _N))
    return lax.dynamic_update_slice(tc_full, sc_out, (0, 0))


def kernel(x, weights, node_order):
    return _run(x, weights, node_order)


# speedup vs baseline: 1.0835x; 1.0835x over previous
import functools

import jax
import jax.numpy as jnp
from jax import lax
from jax.experimental import pallas as pl
from jax.experimental.pallas import tpu as pltpu
from jax.experimental.pallas import tpu_sc as plsc

_BATCH = 16384
_N = 256
_L = 16
_NG = _N // _L
_NC = 1
_NS = 16
_NW = _NC * _NS

_F_SC = 2048
_RPW = _F_SC // _NW
_CH = 64
_NCHUNK = _RPW // _CH
_NPAIR = _NCHUNK // 2

_BLK = 2048
_TC_ROWS = _BATCH - _F_SC


def _sc_body(x_hbm, w_hbm, no_hbm, out_hbm,
             no_v, w_v, xin0, xin1, xout0, xout1,
             sin0, sin1, sout0, sout1):
    wid = lax.axis_index("s") * _NC + lax.axis_index("c")
    base = wid * _RPW
    last = base + _RPW - _CH

    pltpu.sync_copy(no_hbm, no_v)
    pltpu.sync_copy(w_hbm, w_v)
    no_g = [no_v[pl.ds(g * _L, _L)] for g in range(_NG)]
    w_g = [w_v[pl.ds(g * _L, _L)] for g in range(_NG)]

    def start_in(buf, sem, row0):
        pltpu.async_copy(x_hbm.at[pl.ds(row0, _CH)], buf, sem)

    def wait_in(buf, sem):
        pltpu.make_async_copy(x_hbm.at[pl.ds(0, _CH)], buf, sem).wait()

    def start_out(buf, sem, row0):
        pltpu.async_copy(buf, out_hbm.at[pl.ds(row0, _CH)], sem)

    def wait_out(buf, sem):
        pltpu.make_async_copy(buf, out_hbm.at[pl.ds(0, _CH)], sem).wait()

    def compute(xi, xo):
        def row_body(r):
            rvec = jnp.full((_L,), r, jnp.int32)
            for g in range(_NG):
                vals = plsc.load_gather(xi, [rvec, no_g[g]])
                xo[r, pl.ds(g * _L, _L)] = vals * w_g[g]
        plsc.parallel_loop(0, _CH, unroll=1)(row_body)

    start_in(xin0, sin0, base)

    def pair_body(i, _):
        c0 = base + (2 * i) * _CH
        c1 = c0 + _CH
        start_in(xin1, sin1, c1)
        wait_in(xin0, sin0)

        @pl.when(i > 0)
        def _():
            wait_out(xout0, sout0)

        compute(xin0, xout0)
        start_out(xout0, sout0, c0)
        start_in(xin0, sin0, jnp.minimum(c1 + _CH, last))
        wait_in(xin1, sin1)

        @pl.when(i > 0)
        def _():
            wait_out(xout1, sout1)

        compute(xin1, xout1)
        start_out(xout1, sout1, c1)
        return 0

    lax.fori_loop(0, _NPAIR, pair_body, 0)

    wait_in(xin0, sin0)
    wait_out(xout0, sout0)
    wait_out(xout1, sout1)


def _sc_call(x, weights, node_order):
    mesh = plsc.VectorSubcoreMesh(core_axis_name="c", subcore_axis_name="s", num_cores=1)
    k = functools.partial(
        pl.kernel,
        mesh=mesh,
        out_type=jax.ShapeDtypeStruct((_F_SC, _N), jnp.float32),
        compiler_params=pltpu.CompilerParams(needs_layout_passes=False),
        scratch_types=[
            pltpu.VMEM((_N,), jnp.int32),
            pltpu.VMEM((_N,), jnp.float32),
            pltpu.VMEM((_CH, _N), jnp.float32),
            pltpu.VMEM((_CH, _N), jnp.float32),
            pltpu.VMEM((_CH, _N), jnp.float32),
            pltpu.VMEM((_CH, _N), jnp.float32),
            pltpu.SemaphoreType.DMA,
            pltpu.SemaphoreType.DMA,
            pltpu.SemaphoreType.DMA,
            pltpu.SemaphoreType.DMA,
        ],
    )(_sc_body)
    return k(x, weights, node_order)


def _tc_body(no_ref, w_ref, x_ref, o_ref, p_ref):
    @pl.when(pl.program_id(0) == 0)
    def _():
        ids = lax.broadcasted_iota(jnp.int32, (_N, _N), 0)
        p_ref[...] = jnp.where(ids == no_ref[...], w_ref[...],
                               jnp.float32(0.0))

    o_ref[...] = lax.dot_general(
        x_ref[...], p_ref[...], (((1,), (0,)), ((), ())),
        preferred_element_type=jnp.float32)


def _tc_call(x, no2d, w2d):
    nblk = _TC_ROWS // _BLK
    off = _F_SC // _BLK
    return pl.pallas_call(
        _tc_body,
        grid=(nblk,),
        in_specs=[
            pl.BlockSpec((1, _N), lambda i: (0, 0)),
            pl.BlockSpec((1, _N), lambda i: (0, 0)),
            pl.BlockSpec((_BLK, _N), lambda i: (i + off, 0)),
        ],
        out_specs=pl.BlockSpec((_BLK, _N), lambda i: (i + off, 0)),
        out_shape=jax.ShapeDtypeStruct((_BATCH, _N), jnp.float32),
        scratch_shapes=[pltpu.VMEM((_N, _N), jnp.float32)],
    )(no2d, w2d, x)


@jax.jit
def _run(x, weights, node_order):
    sc_out = _sc_call(x, weights, node_order)
    tc_full = _tc_call(x, node_order.reshape(1, _N), weights.reshape(1, _N))
    return lax.dynamic_update_slice(tc_full, sc_out, (0, 0))


def kernel(x, weights, node_order):
    return _run(x, weights, node_order)


# speedup vs baseline: 1.1332x; 1.0459x over previous
import functools

import jax
import jax.numpy as jnp
from jax import lax
from jax.experimental import pallas as pl
from jax.experimental.pallas import tpu as pltpu
from jax.experimental.pallas import tpu_sc as plsc

_BATCH = 16384
_N = 256
_L = 16
_NG = _N // _L
_NC = 1
_NS = 16
_NW = _NC * _NS

_F_SC = 1024
_RPW = _F_SC // _NW
_CH = 32
_NCHUNK = _RPW // _CH
_NPAIR = _NCHUNK // 2

_BLK = 2560
_TC_ROWS = _BATCH - _F_SC


def _sc_body(x_hbm, w_hbm, no_hbm, out_hbm,
             no_v, w_v, xin0, xin1, xout0, xout1,
             sin0, sin1, sout0, sout1):
    wid = lax.axis_index("s") * _NC + lax.axis_index("c")
    obase = wid * _RPW
    base = _TC_ROWS + obase
    last = base + _RPW - _CH

    pltpu.sync_copy(no_hbm, no_v)
    pltpu.sync_copy(w_hbm, w_v)
    no_g = [no_v[pl.ds(g * _L, _L)] for g in range(_NG)]
    w_g = [w_v[pl.ds(g * _L, _L)] for g in range(_NG)]

    def start_in(buf, sem, row0):
        pltpu.async_copy(x_hbm.at[pl.ds(row0, _CH)], buf, sem)

    def wait_in(buf, sem):
        pltpu.make_async_copy(x_hbm.at[pl.ds(0, _CH)], buf, sem).wait()

    def start_out(buf, sem, row0):
        pltpu.async_copy(buf, out_hbm.at[pl.ds(row0, _CH)], sem)

    def wait_out(buf, sem):
        pltpu.make_async_copy(buf, out_hbm.at[pl.ds(0, _CH)], sem).wait()

    def compute(xi, xo):
        def row_body(r):
            rvec = jnp.full((_L,), r, jnp.int32)
            for g in range(_NG):
                vals = plsc.load_gather(xi, [rvec, no_g[g]])
                xo[r, pl.ds(g * _L, _L)] = vals * w_g[g]
        plsc.parallel_loop(0, _CH, unroll=1)(row_body)

    start_in(xin0, sin0, base)

    def pair_body(i, _):
        c0 = base + (2 * i) * _CH
        c1 = c0 + _CH
        o0 = obase + (2 * i) * _CH
        o1 = o0 + _CH
        start_in(xin1, sin1, c1)
        wait_in(xin0, sin0)

        @pl.when(i > 0)
        def _():
            wait_out(xout0, sout0)

        compute(xin0, xout0)
        start_out(xout0, sout0, o0)
        start_in(xin0, sin0, jnp.minimum(c1 + _CH, last))
        wait_in(xin1, sin1)

        @pl.when(i > 0)
        def _():
            wait_out(xout1, sout1)

        compute(xin1, xout1)
        start_out(xout1, sout1, o1)
        return 0

    lax.fori_loop(0, _NPAIR, pair_body, 0)

    wait_in(xin0, sin0)
    wait_out(xout0, sout0)
    wait_out(xout1, sout1)


def _sc_call(x, weights, node_order):
    mesh = plsc.VectorSubcoreMesh(core_axis_name="c", subcore_axis_name="s", num_cores=1)
    k = functools.partial(
        pl.kernel,
        mesh=mesh,
        out_type=jax.ShapeDtypeStruct((_F_SC, _N), jnp.float32),
        compiler_params=pltpu.CompilerParams(needs_layout_passes=False),
        scratch_types=[
            pltpu.VMEM((_N,), jnp.int32),
            pltpu.VMEM((_N,), jnp.float32),
            pltpu.VMEM((_CH, _N), jnp.float32),
            pltpu.VMEM((_CH, _N), jnp.float32),
            pltpu.VMEM((_CH, _N), jnp.float32),
            pltpu.VMEM((_CH, _N), jnp.float32),
            pltpu.SemaphoreType.DMA,
            pltpu.SemaphoreType.DMA,
            pltpu.SemaphoreType.DMA,
            pltpu.SemaphoreType.DMA,
        ],
    )(_sc_body)
    return k(x, weights, node_order)


def _tc_body(no_ref, w_ref, x_ref, o_ref, p_ref):
    @pl.when(pl.program_id(0) == 0)
    def _():
        ids = lax.broadcasted_iota(jnp.int32, (_N, _N), 0)
        p_ref[...] = jnp.where(ids == no_ref[...], w_ref[...],
                               jnp.float32(0.0))

    o_ref[...] = lax.dot_general(
        x_ref[...], p_ref[...], (((1,), (0,)), ((), ())),
        preferred_element_type=jnp.float32)


def _tc_call(x, no2d, w2d):
    nblk = _TC_ROWS // _BLK
    off = 0
    return pl.pallas_call(
        _tc_body,
        grid=(nblk,),
        in_specs=[
            pl.BlockSpec((1, _N), lambda i: (0, 0)),
            pl.BlockSpec((1, _N), lambda i: (0, 0)),
            pl.BlockSpec((_BLK, _N), lambda i: (i, 0)),
        ],
        out_specs=pl.BlockSpec((_BLK, _N), lambda i: (i, 0)),
        out_shape=jax.ShapeDtypeStruct((_BATCH, _N), jnp.float32),
        scratch_shapes=[pltpu.VMEM((_N, _N), jnp.float32)],
    )(no2d, w2d, x)


@jax.jit
def _run(x, weights, node_order):
    sc_out = _sc_call(x, weights, node_order)
    tc_full = _tc_call(x, node_order.reshape(1, _N), weights.reshape(1, _N))
    return lax.dynamic_update_slice(tc_full, sc_out, (_TC_ROWS, 0))


def kernel(x, weights, node_order):
    return _run(x, weights, node_order)
